# baseline (speedup 1.0000x reference)
import functools

import jax
import jax.numpy as jnp
from jax import lax
from jax.experimental import pallas as pl

_IGNORE = -100
_LAMBDA = 0.5
_R = 256


def _tc_body(x_ref, w1_ref, b1_ref, w2_ref, b2_ref, st_ref, mv_ref, out_ref):
    x = x_ref[...]
    h = jnp.tanh(jnp.dot(x, w1_ref[...], preferred_element_type=jnp.float32)
                 + b1_ref[...])
    logits = (jnp.dot(h, w2_ref[...], preferred_element_type=jnp.float32)
              + b2_ref[...])
    t = st_ref[0, 0, :]
    keep = mv_ref[0, 0, :]
    C = logits.shape[1]

    m = jnp.max(logits, axis=1)
    lse = m + jnp.log(jnp.sum(jnp.exp(logits - m[:, None]), axis=1))
    iota_c = lax.broadcasted_iota(jnp.int32, logits.shape, 1)
    xt = jnp.sum(jnp.where(iota_c == t[:, None], logits, 0.0), axis=1)
    validf = (t != _IGNORE).astype(jnp.float32)
    nll_sum = jnp.sum((lse - xt) * validf)
    valid_cnt = jnp.sum(validf)

    pred = jnp.min(jnp.where(logits == m[:, None], iota_c, C), axis=1)
    keepb = keep > 0
    correct = jnp.sum(((pred == t) & keepb).astype(jnp.float32))
    mvalid_cnt = jnp.sum(keepb.astype(jnp.float32))

    rows = lax.broadcasted_iota(jnp.int32, (8, 128), 0)
    cols = lax.broadcasted_iota(jnp.int32, (8, 128), 1)
    r0 = rows == 0
    part = (jnp.where(r0 & (cols == 0), nll_sum, 0.0)
            + jnp.where(r0 & (cols == 1), valid_cnt, 0.0)
            + jnp.where(r0 & (cols == 2), correct, 0.0)
            + jnp.where(r0 & (cols == 3), mvalid_cnt, 0.0))

    @pl.when(pl.program_id(0) == 0)
    def _():
        out_ref[...] = jnp.zeros_like(out_ref)

    out_ref[...] += part


def _realign(token_mask, token_mask_mask, tag, tag_mask):
    B, S = token_mask.shape
    csum = jnp.cumsum(token_mask_mask, axis=-1)
    total = csum[:, -1:]
    keep = ((csum > 1) & (csum <= total - 1) & (token_mask_mask > 0)).astype(jnp.int32)
    tok = token_mask * keep
    M = tag.shape[0] * tag.shape[1]
    flat_tag = tag.reshape(-1)
    flat_tag_mask = tag_mask.reshape(-1)
    rank_tag = jnp.cumsum(flat_tag_mask) - 1
    scatter_idx = jnp.where(flat_tag_mask > 0, rank_tag, M)
    vg = jnp.zeros((M,), dtype=jnp.int32).at[scatter_idx].set(
        flat_tag.astype(jnp.int32), mode='drop')
    flat_tok = tok.reshape(-1)
    rank_tok = jnp.cumsum(flat_tok) - 1
    gathered = vg[jnp.clip(rank_tok, 0, M - 1)]
    sparsed_tag = jnp.where(flat_tok > 0, gathered, jnp.int32(_IGNORE))
    return sparsed_tag.reshape(B, S), keep


def kernel(latent_states, attention_mask, token_mask, token_mask_mask,
           tag, tag_mask, W1, b1, W2, b2):
    B, S, D = latent_states.shape
    H = W1.shape[1]
    C = W2.shape[1]
    N = B * S

    sparsed_tag, keep = _realign(token_mask, token_mask_mask, tag, tag_mask)

    xs = latent_states.reshape(N, D)
    st3 = sparsed_tag.reshape(N // _R, 1, _R)
    mv3 = keep.reshape(N // _R, 1, _R)

    grid = N // _R
    out = pl.pallas_call(
        _tc_body,
        grid=(grid,),
        in_specs=[
            pl.BlockSpec((_R, D), lambda i: (i, 0)),
            pl.BlockSpec((D, H), lambda i: (0, 0)),
            pl.BlockSpec((1, H), lambda i: (0, 0)),
            pl.BlockSpec((H, C), lambda i: (0, 0)),
            pl.BlockSpec((1, C), lambda i: (0, 0)),
            pl.BlockSpec((1, 1, _R), lambda i: (i, 0, 0)),
            pl.BlockSpec((1, 1, _R), lambda i: (i, 0, 0)),
        ],
        out_specs=pl.BlockSpec((8, 128), lambda i: (0, 0)),
        out_shape=jax.ShapeDtypeStruct((8, 128), jnp.float32),
    )(xs, W1, b1.reshape(1, H), W2, b2.reshape(1, C), st3, mv3)

    nll_sum = out[0, 0]
    valid_cnt = out[0, 1]
    correct = out[0, 2]
    mvalid_cnt = out[0, 3]
    cost = _LAMBDA * nll_sum / jnp.maximum(valid_cnt, 1.0)
    acc = correct / jnp.maximum(mvalid_cnt, 1.0)
    return (cost, acc)


# speedup vs baseline: 1.0754x; 1.0754x over previous
import functools

import jax
import jax.numpy as jnp
from jax import lax
from jax.experimental import pallas as pl

_IGNORE = -100
_LAMBDA = 0.5
_R = 512


def _tc_body(x_ref, w1_ref, b1_ref, w2_ref, b2_ref, st_ref, mv_ref, out_ref):
    x = x_ref[...]
    h = jnp.tanh(jnp.dot(x, w1_ref[...], preferred_element_type=jnp.float32)
                 + b1_ref[...])
    logits = (jnp.dot(h, w2_ref[...], preferred_element_type=jnp.float32)
              + b2_ref[...])
    t = st_ref[0, 0, :]
    keep = mv_ref[0, 0, :]
    C = logits.shape[1]

    m = jnp.max(logits, axis=1)
    lse = m + jnp.log(jnp.sum(jnp.exp(logits - m[:, None]), axis=1))
    iota_c = lax.broadcasted_iota(jnp.int32, logits.shape, 1)
    xt = jnp.sum(jnp.where(iota_c == t[:, None], logits, 0.0), axis=1)
    validf = (t != _IGNORE).astype(jnp.float32)
    nll_sum = jnp.sum((lse - xt) * validf)
    valid_cnt = jnp.sum(validf)

    pred = jnp.min(jnp.where(logits == m[:, None], iota_c, C), axis=1)
    keepb = keep > 0
    correct = jnp.sum(((pred == t) & keepb).astype(jnp.float32))
    mvalid_cnt = jnp.sum(keepb.astype(jnp.float32))

    rows = lax.broadcasted_iota(jnp.int32, (8, 128), 0)
    cols = lax.broadcasted_iota(jnp.int32, (8, 128), 1)
    r0 = rows == 0
    part = (jnp.where(r0 & (cols == 0), nll_sum, 0.0)
            + jnp.where(r0 & (cols == 1), valid_cnt, 0.0)
            + jnp.where(r0 & (cols == 2), correct, 0.0)
            + jnp.where(r0 & (cols == 3), mvalid_cnt, 0.0))

    @pl.when(pl.program_id(0) == 0)
    def _():
        out_ref[...] = jnp.zeros_like(out_ref)

    out_ref[...] += part


def _realign(token_mask, token_mask_mask, tag, tag_mask):
    B, S = token_mask.shape
    csum = jnp.cumsum(token_mask_mask, axis=-1)
    total = csum[:, -1:]
    keep = ((csum > 1) & (csum <= total - 1) & (token_mask_mask > 0)).astype(jnp.int32)
    tok = token_mask * keep
    M = tag.shape[0] * tag.shape[1]
    flat_tag = tag.reshape(-1)
    flat_tag_mask = tag_mask.reshape(-1)
    rank_tag = jnp.cumsum(flat_tag_mask) - 1
    scatter_idx = jnp.where(flat_tag_mask > 0, rank_tag, M)
    vg = jnp.zeros((M,), dtype=jnp.int32).at[scatter_idx].set(
        flat_tag.astype(jnp.int32), mode='drop')
    flat_tok = tok.reshape(-1)
    rank_tok = jnp.cumsum(flat_tok) - 1
    gathered = vg[jnp.clip(rank_tok, 0, M - 1)]
    sparsed_tag = jnp.where(flat_tok > 0, gathered, jnp.int32(_IGNORE))
    return sparsed_tag.reshape(B, S), keep


def kernel(latent_states, attention_mask, token_mask, token_mask_mask,
           tag, tag_mask, W1, b1, W2, b2):
    B, S, D = latent_states.shape
    H = W1.shape[1]
    C = W2.shape[1]
    N = B * S

    sparsed_tag, keep = _realign(token_mask, token_mask_mask, tag, tag_mask)

    xs = latent_states.reshape(N, D)
    st3 = sparsed_tag.reshape(N // _R, 1, _R)
    mv3 = keep.reshape(N // _R, 1, _R)

    grid = N // _R
    out = pl.pallas_call(
        _tc_body,
        grid=(grid,),
        in_specs=[
            pl.BlockSpec((_R, D), lambda i: (i, 0)),
            pl.BlockSpec((D, H), lambda i: (0, 0)),
            pl.BlockSpec((1, H), lambda i: (0, 0)),
            pl.BlockSpec((H, C), lambda i: (0, 0)),
            pl.BlockSpec((1, C), lambda i: (0, 0)),
            pl.BlockSpec((1, 1, _R), lambda i: (i, 0, 0)),
            pl.BlockSpec((1, 1, _R), lambda i: (i, 0, 0)),
        ],
        out_specs=pl.BlockSpec((8, 128), lambda i: (0, 0)),
        out_shape=jax.ShapeDtypeStruct((8, 128), jnp.float32),
    )(xs, W1, b1.reshape(1, H), W2, b2.reshape(1, C), st3, mv3)

    nll_sum = out[0, 0]
    valid_cnt = out[0, 1]
    correct = out[0, 2]
    mvalid_cnt = out[0, 3]
    cost = _LAMBDA * nll_sum / jnp.maximum(valid_cnt, 1.0)
    acc = correct / jnp.maximum(mvalid_cnt, 1.0)
    return (cost, acc)


# speedup vs baseline: 1.2991x; 1.2080x over previous
import functools

import jax
import jax.numpy as jnp
from jax import lax
from jax.experimental import pallas as pl
from jax.experimental.pallas import tpu as pltpu

_IGNORE = -100
_LAMBDA = 0.5
_R = 512


def _tc_body(x_ref, w1_ref, b1_ref, w2_ref, b2_ref, st_ref, mv_ref,
             out_ref, h_ref):
    i = pl.program_id(0)

    @pl.when(i == 0)
    def _():
        h_ref[...] = jnp.zeros_like(h_ref)
        out_ref[...] = jnp.zeros_like(out_ref)

    h_prev = h_ref[...]
    logits = (jnp.dot(h_prev, w2_ref[...], preferred_element_type=jnp.float32)
              + b2_ref[...])
    t = st_ref[0, 0, :]
    keep = mv_ref[0, 0, :]
    C = logits.shape[1]

    m = jnp.max(logits, axis=1)
    lse = m + jnp.log(jnp.sum(jnp.exp(logits - m[:, None]), axis=1))
    iota_c = lax.broadcasted_iota(jnp.int32, logits.shape, 1)
    xt = jnp.sum(jnp.where(iota_c == t[:, None], logits, 0.0), axis=1)
    validf = (t != _IGNORE).astype(jnp.float32)
    nll_sum = jnp.sum((lse - xt) * validf)
    valid_cnt = jnp.sum(validf)

    pred = jnp.min(jnp.where(logits == m[:, None], iota_c, C), axis=1)
    keepb = keep > 0
    correct = jnp.sum(((pred == t) & keepb).astype(jnp.float32))
    mvalid_cnt = jnp.sum(keepb.astype(jnp.float32))

    rows = lax.broadcasted_iota(jnp.int32, (8, 128), 0)
    cols = lax.broadcasted_iota(jnp.int32, (8, 128), 1)
    r0 = rows == 0
    part = (jnp.where(r0 & (cols == 0), nll_sum, 0.0)
            + jnp.where(r0 & (cols == 1), valid_cnt, 0.0)
            + jnp.where(r0 & (cols == 2), correct, 0.0)
            + jnp.where(r0 & (cols == 3), mvalid_cnt, 0.0))
    gate = jnp.where(i > 0, 1.0, 0.0).astype(jnp.float32)
    out_ref[...] += part * gate

    h_ref[...] = jnp.tanh(
        jnp.dot(x_ref[...], w1_ref[...], preferred_element_type=jnp.float32)
        + b1_ref[...])


def _realign(token_mask, token_mask_mask, tag, tag_mask):
    B, S = token_mask.shape
    csum = jnp.cumsum(token_mask_mask, axis=-1)
    total = csum[:, -1:]
    keep = ((csum > 1) & (csum <= total - 1) & (token_mask_mask > 0)).astype(jnp.int32)
    tok = token_mask * keep
    M = tag.shape[0] * tag.shape[1]
    flat_tag = tag.reshape(-1)
    flat_tag_mask = tag_mask.reshape(-1)
    rank_tag = jnp.cumsum(flat_tag_mask) - 1
    scatter_idx = jnp.where(flat_tag_mask > 0, rank_tag, M)
    vg = jnp.zeros((M,), dtype=jnp.int32).at[scatter_idx].set(
        flat_tag.astype(jnp.int32), mode='drop')
    flat_tok = tok.reshape(-1)
    rank_tok = jnp.cumsum(flat_tok) - 1
    gathered = vg[jnp.clip(rank_tok, 0, M - 1)]
    sparsed_tag = jnp.where(flat_tok > 0, gathered, jnp.int32(_IGNORE))
    return sparsed_tag.reshape(B, S), keep


def kernel(latent_states, attention_mask, token_mask, token_mask_mask,
           tag, tag_mask, W1, b1, W2, b2):
    B, S, D = latent_states.shape
    H = W1.shape[1]
    C = W2.shape[1]
    N = B * S
    G = N // _R

    sparsed_tag, keep = _realign(token_mask, token_mask_mask, tag, tag_mask)

    xs = latent_states.reshape(N, D)
    st3 = sparsed_tag.reshape(G, 1, _R)
    mv3 = keep.reshape(G, 1, _R)

    def x_map(i):
        return (jnp.minimum(i, G - 1), 0)

    def prev_map(i):
        return (jnp.maximum(i - 1, 0), 0, 0)

    out = pl.pallas_call(
        _tc_body,
        grid=(G + 1,),
        in_specs=[
            pl.BlockSpec((_R, D), x_map),
            pl.BlockSpec((D, H), lambda i: (0, 0)),
            pl.BlockSpec((1, H), lambda i: (0, 0)),
            pl.BlockSpec((H, C), lambda i: (0, 0)),
            pl.BlockSpec((1, C), lambda i: (0, 0)),
            pl.BlockSpec((1, 1, _R), prev_map),
            pl.BlockSpec((1, 1, _R), prev_map),
        ],
        out_specs=pl.BlockSpec((8, 128), lambda i: (0, 0)),
        out_shape=jax.ShapeDtypeStruct((8, 128), jnp.float32),
        scratch_shapes=[pltpu.VMEM((_R, H), jnp.float32)],
    )(xs, W1, b1.reshape(1, H), W2, b2.reshape(1, C), st3, mv3)

    nll_sum = out[0, 0]
    valid_cnt = out[0, 1]
    correct = out[0, 2]
    mvalid_cnt = out[0, 3]
    cost = _LAMBDA * nll_sum / jnp.maximum(valid_cnt, 1.0)
    acc = correct / jnp.maximum(mvalid_cnt, 1.0)
    return (cost, acc)
